# Initial kernel scaffold; baseline (speedup 1.0000x reference)
import jax
import jax.numpy as jnp
from jax.experimental import pallas as pl


def kernel(patches, token_table, pos_table):
    raise NotImplementedError("write your pallas kernel here")


# speedup vs baseline: 3.7569x; 3.7569x over previous
import functools

import jax
import jax.numpy as jnp
from jax import lax
from jax.experimental import pallas as pl
from jax.experimental.pallas import tpu as pltpu
from jax.experimental.pallas import tpu_sc as plsc

_EMBED = 32
_SEQ = 128
_POS_ROWS = 64
_LANES = 16


def _emb_kernel(patches_hbm, tok_hbm, pos_hbm, out_hbm,
                idx_v, rows_v, posrep_v, sem_g):
    info = plsc.get_sparse_core_info()
    num_cores = info.num_cores
    num_workers = num_cores * info.num_subcores
    wid = lax.axis_index("s") * num_cores + lax.axis_index("c")

    batch = patches_hbm.shape[0]
    seqs_per_w = batch // num_workers

    pltpu.sync_copy(pos_hbm, posrep_v.at[pl.ds(0, _POS_ROWS)])
    p0 = posrep_v[_POS_ROWS - 1, pl.ds(0, _LANES)]
    p1 = posrep_v[_POS_ROWS - 1, pl.ds(_LANES, _LANES)]
    for j in range(_POS_ROWS, _SEQ):
        posrep_v[j, pl.ds(0, _LANES)] = p0
        posrep_v[j, pl.ds(_LANES, _LANES)] = p1

    pltpu.sync_copy(patches_hbm.at[pl.ds(wid * seqs_per_w, seqs_per_w)], idx_v)

    def seq_body(j, _):
        pltpu.async_copy(tok_hbm.at[idx_v.at[j]], rows_v, sem_g).wait()

        def add_body(i, _):
            a0 = rows_v[i, pl.ds(0, _LANES)] + posrep_v[i, pl.ds(0, _LANES)]
            a1 = (rows_v[i, pl.ds(_LANES, _LANES)]
                  + posrep_v[i, pl.ds(_LANES, _LANES)])
            rows_v[i, pl.ds(0, _LANES)] = a0
            rows_v[i, pl.ds(_LANES, _LANES)] = a1
            return 0

        lax.fori_loop(0, _SEQ, add_body, 0, unroll=4)

        row0 = (wid * seqs_per_w + j) * _SEQ
        pltpu.sync_copy(rows_v, out_hbm.at[pl.ds(row0, _SEQ)])
        return 0

    lax.fori_loop(0, seqs_per_w, seq_body, 0)


def kernel(patches, token_table, pos_table):
    batch, seq = patches.shape
    vocab, embed = token_table.shape
    idx = patches.astype(jnp.int32)

    mesh = plsc.VectorSubcoreMesh(core_axis_name="c", subcore_axis_name="s")
    n_rows = batch * seq

    run = functools.partial(
        pl.kernel,
        out_type=jax.ShapeDtypeStruct((n_rows, embed), jnp.float32),
        mesh=mesh,
        scratch_types=[
            pltpu.VMEM((batch // 32, seq), jnp.int32),
            pltpu.VMEM((seq, embed), jnp.float32),
            pltpu.VMEM((seq, embed), jnp.float32),
            pltpu.SemaphoreType.DMA,
        ],
        compiler_params=pltpu.CompilerParams(use_tc_tiling_on_sc=False),
    )(_emb_kernel)

    out = run(idx, token_table, pos_table)
    return out.reshape(batch, seq, embed)


# speedup vs baseline: 4.7869x; 1.2742x over previous
import functools

import jax
import jax.numpy as jnp
from jax import lax
from jax.experimental import pallas as pl
from jax.experimental.pallas import tpu as pltpu
from jax.experimental.pallas import tpu_sc as plsc

_EMBED = 32
_SEQ = 128
_POS_ROWS = 64
_LANES = 16
_GRP = 8
_NBUF = 3


def _emb_kernel(patches_hbm, tok_hbm, pos_hbm, out_hbm,
                idx_v, buf0, buf1, buf2, posrep_v, sem_g, sem_s):
    info = plsc.get_sparse_core_info()
    num_cores = info.num_cores
    num_workers = num_cores * info.num_subcores
    wid = lax.axis_index("s") * num_cores + lax.axis_index("c")

    batch = patches_hbm.shape[0]
    seqs_per_w = batch // num_workers
    n_groups = seqs_per_w // _GRP
    bufs = [buf0, buf1, buf2]

    pltpu.sync_copy(pos_hbm, posrep_v.at[pl.ds(0, _POS_ROWS)])
    p0 = posrep_v[_POS_ROWS - 1, pl.ds(0, _LANES)]
    p1 = posrep_v[_POS_ROWS - 1, pl.ds(_LANES, _LANES)]
    for j in range(_POS_ROWS, _SEQ):
        posrep_v[j, pl.ds(0, _LANES)] = p0
        posrep_v[j, pl.ds(_LANES, _LANES)] = p1

    pltpu.sync_copy(patches_hbm.at[pl.ds(wid * seqs_per_w, seqs_per_w)], idx_v)

    def issue_gathers(g, buf):
        return [
            pltpu.async_copy(tok_hbm.at[idx_v.at[g * _GRP + s]],
                             buf.at[pl.ds(s * _SEQ, _SEQ)], sem_g)
            for s in range(_GRP)
        ]

    gathers = issue_gathers(0, bufs[0])
    stores = [None] * n_groups
    for g in range(n_groups):
        cur = bufs[g % _NBUF]
        for c in gathers:
            c.wait()
        if g + 1 < n_groups:
            if g >= _NBUF - 1:
                stores[g - (_NBUF - 1)].wait()
            gathers = issue_gathers(g + 1, bufs[(g + 1) % _NBUF])

        def add_body(i, _, cur=cur):
            q0 = posrep_v[i, pl.ds(0, _LANES)]
            q1 = posrep_v[i, pl.ds(_LANES, _LANES)]
            for s in range(_GRP):
                r = s * _SEQ + i
                a0 = cur[r, pl.ds(0, _LANES)] + q0
                a1 = cur[r, pl.ds(_LANES, _LANES)] + q1
                cur[r, pl.ds(0, _LANES)] = a0
                cur[r, pl.ds(_LANES, _LANES)] = a1
            return 0

        lax.fori_loop(0, _SEQ, add_body, 0)

        row0 = (wid * seqs_per_w + g * _GRP) * _SEQ
        stores[g] = pltpu.async_copy(
            cur, out_hbm.at[pl.ds(row0, _GRP * _SEQ)], sem_s)
    for g in range(n_groups - (_NBUF - 1), n_groups):
        stores[g].wait()


def kernel(patches, token_table, pos_table):
    batch, seq = patches.shape
    vocab, embed = token_table.shape
    idx = patches.astype(jnp.int32)

    mesh = plsc.VectorSubcoreMesh(core_axis_name="c", subcore_axis_name="s")
    n_rows = batch * seq
    buf_t = pltpu.VMEM((_GRP * seq, embed), jnp.float32)

    run = functools.partial(
        pl.kernel,
        out_type=jax.ShapeDtypeStruct((n_rows, embed), jnp.float32),
        mesh=mesh,
        scratch_types=[
            pltpu.VMEM((batch // 32, seq), jnp.int32),
            buf_t, buf_t, buf_t,
            pltpu.VMEM((seq, embed), jnp.float32),
            pltpu.SemaphoreType.DMA,
            pltpu.SemaphoreType.DMA,
        ],
        compiler_params=pltpu.CompilerParams(use_tc_tiling_on_sc=False),
    )(_emb_kernel)

    out = run(idx, token_table, pos_table)
    return out.reshape(batch, seq, embed)
